# baseline (speedup 1.0000x reference)
import numpy as np
import jax
import jax.numpy as jnp
from jax import lax
from jax.experimental import pallas as pl
from jax.experimental.pallas import tpu as pltpu
from jax.experimental.pallas import tpu_sc as plsc

_N_LEVELS = 16
_LOG2_HASH = 19
_HASHMAP_SIZE = 1 << _LOG2_HASH
_HASH_MASK = _HASHMAP_SIZE - 1
_P1 = np.int32(np.uint32(2654435761 & 0xFFFFFFFF))
_P2 = np.int32(805459861)


def _level_resolutions():
    base = np.float32(16.0)
    finest = np.float32(512.0)
    b = np.float32(np.exp((np.log(finest) - np.log(base)) / np.float32(_N_LEVELS - 1)))
    return [np.float32(np.floor(base * np.float32(b ** np.float32(i)))) for i in range(_N_LEVELS)]


_LEVEL_RES = _level_resolutions()
_EMB_SIZES = [min(_HASHMAP_SIZE, int(r) ** 3) for r in _LEVEL_RES]

_P = 256
_N_STAGED = 2
_SPMEM_LEVELS = (2, 3, 4, 5, 6)


def _body(x_ref, *rest):
    emb_refs = rest[:_N_LEVELS]
    out_ref = rest[_N_LEVELS]
    (xv0, xv1, ob0, ob1, idxb0, idxb1, rows0, rows1, fr0, fr1, st0, st1,
     sp2, sp3, sp4, sp5, sp6, sem0, sem1, semo0, semo1, semx0, semx1) = rest[_N_LEVELS + 1:]
    bufs = ((idxb0, rows0, fr0, sem0), (idxb1, rows1, fr1, sem1))
    stages = (st0, st1)
    spmems = {2: sp2, 3: sp3, 4: sp4, 5: sp5, 6: sp6}

    n_pts = x_ref.shape[1]
    per_w = n_pts // 32
    n_chunks = per_w // _P

    wid = lax.axis_index("s") * jnp.int32(2) + lax.axis_index("c")
    base_pt = wid * jnp.int32(per_w)

    iota = lax.iota(jnp.int32, 16)
    zero_f = jnp.zeros((16,), jnp.float32)
    one_f = jnp.ones((16,), jnp.float32)
    half_f = jnp.full((16,), 0.5, jnp.float32)
    one_i = jnp.ones((16,), jnp.int32)

    def coords(xv, o, res_f, fr=None):
        b = [None] * 3
        fv = [None] * 3
        for d in range(3):
            xd = xv[d, pl.ds(o, 16)]
            xc = jnp.minimum(jnp.maximum(xd, zero_f), one_f)
            off = xc * res_f + half_f
            bi = off.astype(jnp.int32)
            fv[d] = off - bi.astype(jnp.float32)
            if fr is not None:
                fr[pl.ds(jnp.int32(d * _P) + o, 16)] = fv[d]
            b[d] = bi
        return b, fv

    def corner_indices(lvl, b):
        res_i = int(_LEVEL_RES[lvl])
        if res_i ** 3 > _HASHMAP_SIZE:
            v10, v11 = b[1], b[1] + one_i
            v20, v21 = b[2], b[2] + one_i
            m10 = v10 * jnp.int32(_P1)
            m11 = v11 * jnp.int32(_P1)
            m20 = v20 * jnp.int32(_P2)
            m21 = v21 * jnp.int32(_P2)
            us = (m10 ^ m20, m10 ^ m21, m11 ^ m20, m11 ^ m21)
            v00, v01 = b[0], b[0] + one_i
            return [((v01 if (c >> 2) else v00) ^ us[c & 3]) & jnp.int32(_HASH_MASK)
                    for c in range(8)]
        res_v = jnp.full((16,), res_i, jnp.int32)
        w = []
        for d in range(3):
            v0 = b[d]
            v1 = b[d] + one_i
            w0 = jnp.where(v0 >= res_v, v0 - res_v, v0)
            w1 = jnp.where(v1 >= res_v, v1 - res_v, v1)
            w.append((w0, w1))
        t10 = w[1][0] * jnp.int32(res_i)
        t11 = w[1][1] * jnp.int32(res_i)
        t20 = w[2][0] * jnp.int32(res_i * res_i)
        t21 = w[2][1] * jnp.int32(res_i * res_i)
        s = (t10 + t20, t10 + t21, t11 + t20, t11 + t21)
        return [(w[0][1] if (c >> 2) else w[0][0]) + s[c & 3] for c in range(8)]

    def lerp8(a, f0, g0, f1, g1, f2, g2):
        e00 = a[0] * g0 + a[4] * f0
        e01 = a[1] * g0 + a[5] * f0
        e10 = a[2] * g0 + a[6] * f0
        e11 = a[3] * g0 + a[7] * f0
        h0 = e00 * g1 + e10 * f1
        h1 = e01 * g1 + e11 * f1
        return h0 * g2 + h1 * f2

    def staged_level(lvl, st, xv, ob):
        res_f = jnp.full((16,), _LEVEL_RES[lvl], jnp.float32)

        @pl.loop(jnp.int32(0), jnp.int32(_P // 16))
        def _st(t):
            o = t * jnp.int32(16)
            b, fv = coords(xv, o, res_f)
            hs = [h + h for h in corner_indices(lvl, b)]
            a0 = [plsc.load_gather(st, [h]) for h in hs]
            a1 = [plsc.load_gather(st, [h + one_i]) for h in hs]
            g0 = one_f - fv[0]
            g1 = one_f - fv[1]
            g2 = one_f - fv[2]
            rb32 = (iota + o) * jnp.int32(32)
            r0 = lerp8(a0, fv[0], g0, fv[1], g1, fv[2], g2)
            r1 = lerp8(a1, fv[0], g0, fv[1], g1, fv[2], g2)
            plsc.store_scatter(ob, [rb32 + jnp.int32(2 * lvl)], r0)
            plsc.store_scatter(ob, [rb32 + jnp.int32(2 * lvl + 1)], r1)

    def pass1(lvl, idxb, fr, xv):
        res_f = jnp.full((16,), _LEVEL_RES[lvl], jnp.float32)

        @pl.loop(jnp.int32(0), jnp.int32(_P // 16))
        def _idx(t):
            o = t * jnp.int32(16)
            b, _ = coords(xv, o, res_f, fr)
            hs = corner_indices(lvl, b)
            for c in range(8):
                h2 = hs[c] + hs[c]
                idxb[pl.ds(jnp.int32(c * _P) + o, 16)] = h2
                idxb[pl.ds(jnp.int32((8 + c) * _P) + o, 16)] = h2 + one_i

    def interp(lvl, rows, fr, ob):
        @pl.loop(jnp.int32(0), jnp.int32(_P // 16))
        def _interp(t):
            o = t * jnp.int32(16)
            f0 = fr[pl.ds(o, 16)]
            f1 = fr[pl.ds(jnp.int32(_P) + o, 16)]
            f2 = fr[pl.ds(jnp.int32(2 * _P) + o, 16)]
            g0 = one_f - f0
            g1 = one_f - f1
            g2 = one_f - f2
            rb32 = (iota + o) * jnp.int32(32)
            a0 = [rows[pl.ds(jnp.int32(c * _P) + o, 16)] for c in range(8)]
            a1 = [rows[pl.ds(jnp.int32((8 + c) * _P) + o, 16)] for c in range(8)]
            r0 = lerp8(a0, f0, g0, f1, g1, f2, g2)
            r1 = lerp8(a1, f0, g0, f1, g1, f2, g2)
            plsc.store_scatter(ob, [rb32 + jnp.int32(2 * lvl)], r0)
            plsc.store_scatter(ob, [rb32 + jnp.int32(2 * lvl + 1)], r1)

    for l in range(_N_STAGED):
        pltpu.sync_copy(emb_refs[l], stages[l])

    sid = lax.axis_index("s")
    for l in _SPMEM_LEVELS:
        @pl.when(sid == jnp.int32(l))
        def _copy():
            pltpu.sync_copy(emb_refs[l], spmems[l])
    plsc.subcore_barrier()

    def process_chunk(c, xv, ob):
        descs = [None] * _N_LEVELS

        def launch(lvl):
            idxb, rows, fr, sem = bufs[lvl % 2]
            pass1(lvl, idxb, fr, xv)
            src = spmems[lvl] if lvl in _SPMEM_LEVELS else emb_refs[lvl]
            d = pltpu.make_async_copy(src.at[idxb], rows, sem)
            d.start()
            descs[lvl] = d

        launch(_N_STAGED)
        for l in range(_N_STAGED):
            staged_level(l, stages[l], xv, ob)
        for lvl in range(_N_STAGED, _N_LEVELS):
            if lvl + 1 < _N_LEVELS:
                launch(lvl + 1)
            descs[lvl].wait()
            _, rows, fr, _ = bufs[lvl % 2]
            interp(lvl, rows, fr, ob)

    pltpu.sync_copy(x_ref.at[:, pl.ds(base_pt, _P)], xv0)

    @pl.loop(jnp.int32(0), jnp.int32(n_chunks), step=jnp.int32(2))
    def _chunk2(ci):
        for par in (0, 1):
            xv = (xv0, xv1)[par]
            xvn = (xv1, xv0)[par]
            obx = (ob0, ob1)[par]
            semo = (semo0, semo1)[par]
            semx = (semx0, semx1)[par]
            semxn = (semx1, semx0)[par]
            c = ci + jnp.int32(par)
            col = base_pt + c * jnp.int32(_P)
            nxt = c + jnp.int32(1)

            @pl.when(nxt < jnp.int32(n_chunks))
            def _prefetch():
                pltpu.make_async_copy(
                    x_ref.at[:, pl.ds(base_pt + nxt * jnp.int32(_P), _P)],
                    xvn, semxn).start()

            @pl.when(c > jnp.int32(0))
            def _wait_x():
                pltpu.make_async_copy(
                    x_ref.at[:, pl.ds(col, _P)], xv, semx).wait()

            @pl.when(c >= jnp.int32(2))
            def _wait_out():
                pltpu.make_async_copy(
                    obx, out_ref.at[pl.ds(col * jnp.int32(32), _P * 32)],
                    semo).wait()

            process_chunk(c, xv, obx)
            pltpu.make_async_copy(
                obx, out_ref.at[pl.ds(col * jnp.int32(32), _P * 32)],
                semo).start()

    for obx, semo in ((ob0, semo0), (ob1, semo1)):
        pltpu.make_async_copy(
            obx, out_ref.at[pl.ds(base_pt * jnp.int32(32), _P * 32)],
            semo).wait()


def kernel(x, emb0, emb1, emb2, emb3, emb4, emb5, emb6, emb7, emb8,
           emb9, emb10, emb11, emb12, emb13, emb14, emb15):
    embs = [emb0, emb1, emb2, emb3, emb4, emb5, emb6, emb7, emb8,
            emb9, emb10, emb11, emb12, emb13, emb14, emb15]
    n = x.shape[0]
    x_t = x.T
    embs = [e.reshape(-1) for e in embs]

    mesh = plsc.VectorSubcoreMesh(core_axis_name="c", subcore_axis_name="s")
    f = pl.kernel(
        _body,
        out_type=jax.ShapeDtypeStruct((n * 2 * _N_LEVELS,), jnp.float32),
        mesh=mesh,
        compiler_params=pltpu.CompilerParams(needs_layout_passes=False),
        scratch_types=[
            pltpu.VMEM((3, _P), jnp.float32),
            pltpu.VMEM((3, _P), jnp.float32),
            pltpu.VMEM((_P * 2 * _N_LEVELS,), jnp.float32),
            pltpu.VMEM((_P * 2 * _N_LEVELS,), jnp.float32),
            pltpu.VMEM((16 * _P,), jnp.int32),
            pltpu.VMEM((16 * _P,), jnp.int32),
            pltpu.VMEM((16 * _P,), jnp.float32),
            pltpu.VMEM((16 * _P,), jnp.float32),
            pltpu.VMEM((3 * _P,), jnp.float32),
            pltpu.VMEM((3 * _P,), jnp.float32),
            pltpu.VMEM((2 * _EMB_SIZES[0],), jnp.float32),
            pltpu.VMEM((2 * _EMB_SIZES[1],), jnp.float32),
            pltpu.VMEM_SHARED((2 * _EMB_SIZES[2],), jnp.float32),
            pltpu.VMEM_SHARED((2 * _EMB_SIZES[3],), jnp.float32),
            pltpu.VMEM_SHARED((2 * _EMB_SIZES[4],), jnp.float32),
            pltpu.VMEM_SHARED((2 * _EMB_SIZES[5],), jnp.float32),
            pltpu.VMEM_SHARED((2 * _EMB_SIZES[6],), jnp.float32),
            pltpu.SemaphoreType.DMA,
            pltpu.SemaphoreType.DMA,
            pltpu.SemaphoreType.DMA,
            pltpu.SemaphoreType.DMA,
            pltpu.SemaphoreType.DMA,
            pltpu.SemaphoreType.DMA,
        ],
    )
    return f(x_t, *embs).reshape(n, 2 * _N_LEVELS)


# speedup vs baseline: 1.8861x; 1.8861x over previous
import numpy as np
import jax
import jax.numpy as jnp
from jax import lax
from jax.experimental import pallas as pl
from jax.experimental.pallas import tpu as pltpu
from jax.experimental.pallas import tpu_sc as plsc

_N_LEVELS = 16
_LOG2_HASH = 19
_HASHMAP_SIZE = 1 << _LOG2_HASH
_HASH_MASK = _HASHMAP_SIZE - 1
_P1 = np.int32(np.uint32(2654435761 & 0xFFFFFFFF))
_P2 = np.int32(805459861)


def _level_resolutions():
    base = np.float32(16.0)
    finest = np.float32(512.0)
    b = np.float32(np.exp((np.log(finest) - np.log(base)) / np.float32(_N_LEVELS - 1)))
    return [np.float32(np.floor(base * np.float32(b ** np.float32(i)))) for i in range(_N_LEVELS)]


_LEVEL_RES = _level_resolutions()
_EMB_SIZES = [min(_HASHMAP_SIZE, int(r) ** 3) for r in _LEVEL_RES]

_P = 256
_N_STAGED = 2
_SPMEM_LEVELS = (2, 3, 4, 5, 6)


def _body(x_ref, *rest):
    emb_refs = rest[:_N_LEVELS]
    out_ref = rest[_N_LEVELS]
    (xv0, xv1, ob0, ob1, idxb0, idxb1, rows0, rows1, fr0, fr1, st0, st1,
     sp2, sp3, sp4, sp5, sp6, sem0, sem1, semo0, semo1, semx0, semx1) = rest[_N_LEVELS + 1:]
    bufs = ((idxb0, rows0, fr0, sem0), (idxb1, rows1, fr1, sem1))
    stages = (st0, st1)
    spmems = {2: sp2, 3: sp3, 4: sp4, 5: sp5, 6: sp6}

    n_pts = x_ref.shape[1]
    per_w = n_pts // 32
    n_chunks = per_w // _P

    wid = lax.axis_index("s") * jnp.int32(2) + lax.axis_index("c")
    base_pt = wid * jnp.int32(per_w)

    iota = lax.iota(jnp.int32, 16)
    zero_f = jnp.zeros((16,), jnp.float32)
    one_f = jnp.ones((16,), jnp.float32)
    half_f = jnp.full((16,), 0.5, jnp.float32)
    one_i = jnp.ones((16,), jnp.int32)

    def coords(xv, o, res_f, fr=None):
        b = [None] * 3
        fv = [None] * 3
        for d in range(3):
            xd = xv[d, pl.ds(o, 16)]
            xc = jnp.minimum(jnp.maximum(xd, zero_f), one_f)
            off = xc * res_f + half_f
            bi = off.astype(jnp.int32)
            fv[d] = off - bi.astype(jnp.float32)
            if fr is not None:
                fr[pl.ds(jnp.int32(d * _P) + o, 16)] = fv[d]
            b[d] = bi
        return b, fv

    def corner_indices(lvl, b):
        res_i = int(_LEVEL_RES[lvl])
        if res_i ** 3 > _HASHMAP_SIZE:
            v10, v11 = b[1], b[1] + one_i
            v20, v21 = b[2], b[2] + one_i
            m10 = v10 * jnp.int32(_P1)
            m11 = v11 * jnp.int32(_P1)
            m20 = v20 * jnp.int32(_P2)
            m21 = v21 * jnp.int32(_P2)
            us = (m10 ^ m20, m10 ^ m21, m11 ^ m20, m11 ^ m21)
            v00, v01 = b[0], b[0] + one_i
            return [((v01 if (c >> 2) else v00) ^ us[c & 3]) & jnp.int32(_HASH_MASK)
                    for c in range(8)]
        res_v = jnp.full((16,), res_i, jnp.int32)
        w = []
        for d in range(3):
            v0 = b[d]
            v1 = b[d] + one_i
            w0 = jnp.where(v0 >= res_v, v0 - res_v, v0)
            w1 = jnp.where(v1 >= res_v, v1 - res_v, v1)
            w.append((w0, w1))
        t10 = w[1][0] * jnp.int32(res_i)
        t11 = w[1][1] * jnp.int32(res_i)
        t20 = w[2][0] * jnp.int32(res_i * res_i)
        t21 = w[2][1] * jnp.int32(res_i * res_i)
        s = (t10 + t20, t10 + t21, t11 + t20, t11 + t21)
        return [(w[0][1] if (c >> 2) else w[0][0]) + s[c & 3] for c in range(8)]

    def lerp8(a, f0, g0, f1, g1, f2, g2):
        e00 = a[0] * g0 + a[4] * f0
        e01 = a[1] * g0 + a[5] * f0
        e10 = a[2] * g0 + a[6] * f0
        e11 = a[3] * g0 + a[7] * f0
        h0 = e00 * g1 + e10 * f1
        h1 = e01 * g1 + e11 * f1
        return h0 * g2 + h1 * f2

    def staged_level(lvl, st, xv, ob):
        res_f = jnp.full((16,), _LEVEL_RES[lvl], jnp.float32)

        sz_v = jnp.full((16,), _EMB_SIZES[lvl], jnp.int32)

        @pl.loop(jnp.int32(0), jnp.int32(_P // 16))
        def _st(t):
            o = t * jnp.int32(16)
            b, fv = coords(xv, o, res_f)
            hs = corner_indices(lvl, b)
            a0 = [plsc.load_gather(st, [h]) for h in hs]
            a1 = [plsc.load_gather(st, [h + sz_v]) for h in hs]
            g0 = one_f - fv[0]
            g1 = one_f - fv[1]
            g2 = one_f - fv[2]
            rb32 = (iota + o) * jnp.int32(32)
            r0 = lerp8(a0, fv[0], g0, fv[1], g1, fv[2], g2)
            r1 = lerp8(a1, fv[0], g0, fv[1], g1, fv[2], g2)
            plsc.store_scatter(ob, [rb32 + jnp.int32(2 * lvl)], r0)
            plsc.store_scatter(ob, [rb32 + jnp.int32(2 * lvl + 1)], r1)

    def pass1(lvl, idxb, fr, xv):
        res_f = jnp.full((16,), _LEVEL_RES[lvl], jnp.float32)

        sz_v = jnp.full((16,), _EMB_SIZES[lvl], jnp.int32)

        @pl.loop(jnp.int32(0), jnp.int32(_P // 16))
        def _idx(t):
            o = t * jnp.int32(16)
            b, _ = coords(xv, o, res_f, fr)
            hs = corner_indices(lvl, b)
            for c in range(8):
                idxb[pl.ds(jnp.int32(c * _P) + o, 16)] = hs[c]
                idxb[pl.ds(jnp.int32((8 + c) * _P) + o, 16)] = hs[c] + sz_v

    def interp(lvl, rows, fr, ob):
        @pl.loop(jnp.int32(0), jnp.int32(_P // 16))
        def _interp(t):
            o = t * jnp.int32(16)
            f0 = fr[pl.ds(o, 16)]
            f1 = fr[pl.ds(jnp.int32(_P) + o, 16)]
            f2 = fr[pl.ds(jnp.int32(2 * _P) + o, 16)]
            g0 = one_f - f0
            g1 = one_f - f1
            g2 = one_f - f2
            rb32 = (iota + o) * jnp.int32(32)
            a0 = [rows[pl.ds(jnp.int32(c * _P) + o, 16)] for c in range(8)]
            a1 = [rows[pl.ds(jnp.int32((8 + c) * _P) + o, 16)] for c in range(8)]
            r0 = lerp8(a0, f0, g0, f1, g1, f2, g2)
            r1 = lerp8(a1, f0, g0, f1, g1, f2, g2)
            plsc.store_scatter(ob, [rb32 + jnp.int32(2 * lvl)], r0)
            plsc.store_scatter(ob, [rb32 + jnp.int32(2 * lvl + 1)], r1)

    for l in range(_N_STAGED):
        pltpu.sync_copy(emb_refs[l], stages[l])

    sid = lax.axis_index("s")
    for l in _SPMEM_LEVELS:
        @pl.when(sid == jnp.int32(l))
        def _copy():
            pltpu.sync_copy(emb_refs[l], spmems[l])
    plsc.subcore_barrier()

    def process_chunk(c, xv, ob):
        descs = [None] * _N_LEVELS

        def launch(lvl):
            idxb, rows, fr, sem = bufs[lvl % 2]
            pass1(lvl, idxb, fr, xv)
            src = spmems[lvl] if lvl in _SPMEM_LEVELS else emb_refs[lvl]
            d = pltpu.make_async_copy(src.at[idxb], rows, sem)
            d.start()
            descs[lvl] = d

        launch(_N_STAGED)
        for l in range(_N_STAGED):
            staged_level(l, stages[l], xv, ob)
        for lvl in range(_N_STAGED, _N_LEVELS):
            if lvl + 1 < _N_LEVELS:
                launch(lvl + 1)
            descs[lvl].wait()
            _, rows, fr, _ = bufs[lvl % 2]
            interp(lvl, rows, fr, ob)

    pltpu.sync_copy(x_ref.at[:, pl.ds(base_pt, _P)], xv0)

    @pl.loop(jnp.int32(0), jnp.int32(n_chunks), step=jnp.int32(2))
    def _chunk2(ci):
        for par in (0, 1):
            xv = (xv0, xv1)[par]
            xvn = (xv1, xv0)[par]
            obx = (ob0, ob1)[par]
            semo = (semo0, semo1)[par]
            semx = (semx0, semx1)[par]
            semxn = (semx1, semx0)[par]
            c = ci + jnp.int32(par)
            col = base_pt + c * jnp.int32(_P)
            nxt = c + jnp.int32(1)

            @pl.when(nxt < jnp.int32(n_chunks))
            def _prefetch():
                pltpu.make_async_copy(
                    x_ref.at[:, pl.ds(base_pt + nxt * jnp.int32(_P), _P)],
                    xvn, semxn).start()

            @pl.when(c > jnp.int32(0))
            def _wait_x():
                pltpu.make_async_copy(
                    x_ref.at[:, pl.ds(col, _P)], xv, semx).wait()

            @pl.when(c >= jnp.int32(2))
            def _wait_out():
                pltpu.make_async_copy(
                    obx, out_ref.at[pl.ds(col * jnp.int32(32), _P * 32)],
                    semo).wait()

            process_chunk(c, xv, obx)
            pltpu.make_async_copy(
                obx, out_ref.at[pl.ds(col * jnp.int32(32), _P * 32)],
                semo).start()

    for obx, semo in ((ob0, semo0), (ob1, semo1)):
        pltpu.make_async_copy(
            obx, out_ref.at[pl.ds(base_pt * jnp.int32(32), _P * 32)],
            semo).wait()


def kernel(x, emb0, emb1, emb2, emb3, emb4, emb5, emb6, emb7, emb8,
           emb9, emb10, emb11, emb12, emb13, emb14, emb15):
    embs = [emb0, emb1, emb2, emb3, emb4, emb5, emb6, emb7, emb8,
            emb9, emb10, emb11, emb12, emb13, emb14, emb15]
    n = x.shape[0]
    x_t = x.T
    embs = [e.T.reshape(-1) for e in embs]

    mesh = plsc.VectorSubcoreMesh(core_axis_name="c", subcore_axis_name="s")
    f = pl.kernel(
        _body,
        out_type=jax.ShapeDtypeStruct((n * 2 * _N_LEVELS,), jnp.float32),
        mesh=mesh,
        compiler_params=pltpu.CompilerParams(needs_layout_passes=False),
        scratch_types=[
            pltpu.VMEM((3, _P), jnp.float32),
            pltpu.VMEM((3, _P), jnp.float32),
            pltpu.VMEM((_P * 2 * _N_LEVELS,), jnp.float32),
            pltpu.VMEM((_P * 2 * _N_LEVELS,), jnp.float32),
            pltpu.VMEM((16 * _P,), jnp.int32),
            pltpu.VMEM((16 * _P,), jnp.int32),
            pltpu.VMEM((16 * _P,), jnp.float32),
            pltpu.VMEM((16 * _P,), jnp.float32),
            pltpu.VMEM((3 * _P,), jnp.float32),
            pltpu.VMEM((3 * _P,), jnp.float32),
            pltpu.VMEM((2 * _EMB_SIZES[0],), jnp.float32),
            pltpu.VMEM((2 * _EMB_SIZES[1],), jnp.float32),
            pltpu.VMEM_SHARED((2 * _EMB_SIZES[2],), jnp.float32),
            pltpu.VMEM_SHARED((2 * _EMB_SIZES[3],), jnp.float32),
            pltpu.VMEM_SHARED((2 * _EMB_SIZES[4],), jnp.float32),
            pltpu.VMEM_SHARED((2 * _EMB_SIZES[5],), jnp.float32),
            pltpu.VMEM_SHARED((2 * _EMB_SIZES[6],), jnp.float32),
            pltpu.SemaphoreType.DMA,
            pltpu.SemaphoreType.DMA,
            pltpu.SemaphoreType.DMA,
            pltpu.SemaphoreType.DMA,
            pltpu.SemaphoreType.DMA,
            pltpu.SemaphoreType.DMA,
        ],
    )
    return f(x_t, *embs).reshape(n, 2 * _N_LEVELS)
